# Initial kernel scaffold; baseline (speedup 1.0000x reference)
import jax
import jax.numpy as jnp
from jax.experimental import pallas as pl


def kernel(gI, gT, W_i, W_t, ln_i_g, ln_i_b, ln_t_g, ln_t_b):
    raise NotImplementedError("write your pallas kernel here")


# speedup vs baseline: 10.1557x; 10.1557x over previous
import functools

import jax
import jax.numpy as jnp
from jax.experimental import pallas as pl
from jax.experimental.pallas import tpu as pltpu

TAU = 0.2
TOPK = 8
ALPHA = 0.3
B = 4096
D = 128
BM = 256

_NEG = -3.0e38


def _norm_rows(x):
    ss = jnp.sum(x * x, axis=1, keepdims=True)
    return x * jax.lax.rsqrt(jnp.maximum(ss, 1e-24))


def _layer_norm(y, gamma, beta):
    mu = jnp.mean(y, axis=1, keepdims=True)
    var = jnp.mean((y - mu) * (y - mu), axis=1, keepdims=True)
    return (y - mu) * jax.lax.rsqrt(var + 1e-5) * gamma + beta


def _topk_softmax_msg(s, P, g_raw, gamma, beta):
    m1 = jnp.max(s, axis=1, keepdims=True)
    x = s
    m = m1
    for _ in range(TOPK - 1):
        x = jnp.where(x >= m, _NEG, x)
        m = jnp.max(x, axis=1, keepdims=True)
    thr = m
    e = jnp.where(s >= thr, jnp.exp(s - m1), 0.0)
    z = jnp.sum(e, axis=1, keepdims=True)
    msg = jax.lax.dot_general(
        e.astype(jnp.bfloat16), P,
        (((1,), (0,)), ((), ())),
        preferred_element_type=jnp.float32,
    ) / z
    return _layer_norm(g_raw + ALPHA * msg, gamma, beta)


def _body(gI_blk, gT_blk, gI_full, gT_full, W_i, W_t,
          ln_i_g, ln_i_b, ln_t_g, ln_t_b,
          S_ref, gI2_ref, gT2_ref,
          gin, gtn, Pi, Pt):
    i = pl.program_id(0)

    @pl.when(i == 0)
    def _init():
        gI = gI_full[...]
        gT = gT_full[...]
        gin[...] = _norm_rows(gI).astype(jnp.bfloat16)
        gtn[...] = _norm_rows(gT).astype(jnp.bfloat16)
        Pi[...] = jax.lax.dot_general(
            gI.astype(jnp.bfloat16), W_i[...].astype(jnp.bfloat16),
            (((1,), (1,)), ((), ())),
            preferred_element_type=jnp.float32).astype(jnp.bfloat16)
        Pt[...] = jax.lax.dot_general(
            gT.astype(jnp.bfloat16), W_t[...].astype(jnp.bfloat16),
            (((1,), (1,)), ((), ())),
            preferred_element_type=jnp.float32).astype(jnp.bfloat16)

    inv_tau = jnp.float32(1.0 / TAU)

    gib = _norm_rows(gI_blk[...]).astype(jnp.bfloat16)
    s = jax.lax.dot_general(
        gib, gtn[...],
        (((1,), (1,)), ((), ())),
        preferred_element_type=jnp.float32) * inv_tau
    S_ref[...] = s
    gI2_ref[...] = _topk_softmax_msg(
        s, Pt[...], gI_blk[...], ln_i_g[...], ln_i_b[...])

    gtb = _norm_rows(gT_blk[...]).astype(jnp.bfloat16)
    st = jax.lax.dot_general(
        gtb, gin[...],
        (((1,), (1,)), ((), ())),
        preferred_element_type=jnp.float32) * inv_tau
    gT2_ref[...] = _topk_softmax_msg(
        st, Pi[...], gT_blk[...], ln_t_g[...], ln_t_b[...])


@jax.jit
def kernel(gI, gT, W_i, W_t, ln_i_g, ln_i_b, ln_t_g, ln_t_b):
    grid = (B // BM,)
    blk = lambda i: (i, 0)
    full = lambda i: (0, 0)
    out_shapes = (
        jax.ShapeDtypeStruct((B, B), jnp.float32),
        jax.ShapeDtypeStruct((B, D), jnp.float32),
        jax.ShapeDtypeStruct((B, D), jnp.float32),
    )
    S, gI2, gT2 = pl.pallas_call(
        _body,
        grid=grid,
        in_specs=[
            pl.BlockSpec((BM, D), blk),
            pl.BlockSpec((BM, D), blk),
            pl.BlockSpec((B, D), full),
            pl.BlockSpec((B, D), full),
            pl.BlockSpec((D, D), full),
            pl.BlockSpec((D, D), full),
            pl.BlockSpec((1, D), full),
            pl.BlockSpec((1, D), full),
            pl.BlockSpec((1, D), full),
            pl.BlockSpec((1, D), full),
        ],
        out_specs=(
            pl.BlockSpec((BM, B), blk),
            pl.BlockSpec((BM, D), blk),
            pl.BlockSpec((BM, D), blk),
        ),
        out_shape=out_shapes,
        scratch_shapes=[
            pltpu.VMEM((B, D), jnp.bfloat16),
            pltpu.VMEM((B, D), jnp.bfloat16),
            pltpu.VMEM((B, D), jnp.bfloat16),
            pltpu.VMEM((B, D), jnp.bfloat16),
        ],
    )(gI, gT, gI, gT, W_i, W_t,
      ln_i_g.reshape(1, D), ln_i_b.reshape(1, D),
      ln_t_g.reshape(1, D), ln_t_b.reshape(1, D))
    return (gI2, gT2, S)


# speedup vs baseline: 11.2693x; 1.1097x over previous
import functools

import jax
import jax.numpy as jnp
from jax.experimental import pallas as pl
from jax.experimental.pallas import tpu as pltpu

TAU = 0.2
TOPK = 8
ALPHA = 0.3
B = 4096
D = 128
BM = 256

_NEG = -3.0e38


def _norm_rows(x):
    ss = jnp.sum(x * x, axis=1, keepdims=True)
    return x * jax.lax.rsqrt(jnp.maximum(ss, 1e-24))


def _layer_norm(y, gamma, beta):
    mu = jnp.mean(y, axis=1, keepdims=True)
    var = jnp.mean((y - mu) * (y - mu), axis=1, keepdims=True)
    return (y - mu) * jax.lax.rsqrt(var + 1e-5) * gamma + beta


def _topk_softmax_msg(s, P, g_raw, gamma, beta):
    sb = s.astype(jnp.bfloat16)
    m1 = jnp.max(sb, axis=1, keepdims=True)
    x = sb
    m = m1
    neg = jnp.bfloat16(_NEG)
    for _ in range(TOPK - 1):
        x = jnp.where(x >= m, neg, x)
        m = jnp.max(x, axis=1, keepdims=True)
    thr = m
    e = jnp.where(sb >= thr, jnp.exp(s - m1.astype(jnp.float32)), 0.0)
    z = jnp.sum(e, axis=1, keepdims=True)
    msg = jax.lax.dot_general(
        e.astype(jnp.bfloat16), P,
        (((1,), (0,)), ((), ())),
        preferred_element_type=jnp.float32,
    ) / z
    return _layer_norm(g_raw + ALPHA * msg, gamma, beta)


def _body(gI_blk, gT_blk, gI_full, gT_full, W_i, W_t,
          ln_i_g, ln_i_b, ln_t_g, ln_t_b,
          S_ref, gI2_ref, gT2_ref,
          gin, gtn, Pi, Pt):
    i = pl.program_id(0)

    @pl.when(i == 0)
    def _init():
        gI = gI_full[...]
        gT = gT_full[...]
        gin[...] = _norm_rows(gI).astype(jnp.bfloat16)
        gtn[...] = _norm_rows(gT).astype(jnp.bfloat16)
        Pi[...] = jax.lax.dot_general(
            gI.astype(jnp.bfloat16), W_i[...].astype(jnp.bfloat16),
            (((1,), (1,)), ((), ())),
            preferred_element_type=jnp.float32).astype(jnp.bfloat16)
        Pt[...] = jax.lax.dot_general(
            gT.astype(jnp.bfloat16), W_t[...].astype(jnp.bfloat16),
            (((1,), (1,)), ((), ())),
            preferred_element_type=jnp.float32).astype(jnp.bfloat16)

    inv_tau = jnp.float32(1.0 / TAU)

    gib = _norm_rows(gI_blk[...]).astype(jnp.bfloat16)
    s = jax.lax.dot_general(
        gib, gtn[...],
        (((1,), (1,)), ((), ())),
        preferred_element_type=jnp.float32) * inv_tau
    S_ref[...] = s
    gI2_ref[...] = _topk_softmax_msg(
        s, Pt[...], gI_blk[...], ln_i_g[...], ln_i_b[...])

    gtb = _norm_rows(gT_blk[...]).astype(jnp.bfloat16)
    st = jax.lax.dot_general(
        gtb, gin[...],
        (((1,), (1,)), ((), ())),
        preferred_element_type=jnp.float32) * inv_tau
    gT2_ref[...] = _topk_softmax_msg(
        st, Pi[...], gT_blk[...], ln_t_g[...], ln_t_b[...])


@jax.jit
def kernel(gI, gT, W_i, W_t, ln_i_g, ln_i_b, ln_t_g, ln_t_b):
    grid = (B // BM,)
    blk = lambda i: (i, 0)
    full = lambda i: (0, 0)
    out_shapes = (
        jax.ShapeDtypeStruct((B, B), jnp.float32),
        jax.ShapeDtypeStruct((B, D), jnp.float32),
        jax.ShapeDtypeStruct((B, D), jnp.float32),
    )
    S, gI2, gT2 = pl.pallas_call(
        _body,
        grid=grid,
        in_specs=[
            pl.BlockSpec((BM, D), blk),
            pl.BlockSpec((BM, D), blk),
            pl.BlockSpec((B, D), full),
            pl.BlockSpec((B, D), full),
            pl.BlockSpec((D, D), full),
            pl.BlockSpec((D, D), full),
            pl.BlockSpec((1, D), full),
            pl.BlockSpec((1, D), full),
            pl.BlockSpec((1, D), full),
            pl.BlockSpec((1, D), full),
        ],
        out_specs=(
            pl.BlockSpec((BM, B), blk),
            pl.BlockSpec((BM, D), blk),
            pl.BlockSpec((BM, D), blk),
        ),
        out_shape=out_shapes,
        scratch_shapes=[
            pltpu.VMEM((B, D), jnp.bfloat16),
            pltpu.VMEM((B, D), jnp.bfloat16),
            pltpu.VMEM((B, D), jnp.bfloat16),
            pltpu.VMEM((B, D), jnp.bfloat16),
        ],
    )(gI, gT, gI, gT, W_i, W_t,
      ln_i_g.reshape(1, D), ln_i_b.reshape(1, D),
      ln_t_g.reshape(1, D), ln_t_b.reshape(1, D))
    return (gI2, gT2, S)


# speedup vs baseline: 14.5357x; 1.2899x over previous
import functools

import jax
import jax.numpy as jnp
from jax.experimental import pallas as pl
from jax.experimental.pallas import tpu as pltpu

TAU = 0.2
TOPK = 8
ALPHA = 0.3
B = 4096
D = 128
BM = 256

_NEG = -3.0e38


def _norm_rows(x):
    ss = jnp.sum(x * x, axis=1, keepdims=True)
    return x * jax.lax.rsqrt(jnp.maximum(ss, 1e-24))


def _layer_norm(y, gamma, beta):
    mu = jnp.mean(y, axis=1, keepdims=True)
    var = jnp.mean((y - mu) * (y - mu), axis=1, keepdims=True)
    return (y - mu) * jax.lax.rsqrt(var + 1e-5) * gamma + beta


_SORT8_STAGES = (
    ((0, 1), (2, 3), (4, 5), (6, 7)),
    ((0, 2), (1, 3), (4, 6), (5, 7)),
    ((1, 2), (5, 6)),
    ((0, 4), (1, 5), (2, 6), (3, 7)),
    ((2, 4), (3, 5)),
    ((1, 2), (3, 4), (5, 6)),
)
_BITONIC8_STAGES = (
    ((0, 4), (1, 5), (2, 6), (3, 7)),
    ((0, 2), (1, 3), (4, 6), (5, 7)),
    ((0, 1), (2, 3), (4, 5), (6, 7)),
)


def _apply_net(v, stages):
    v = list(v)
    for stage in stages:
        for i, j in stage:
            hi = jnp.maximum(v[i], v[j])
            lo = jnp.minimum(v[i], v[j])
            v[i], v[j] = hi, lo
    return v


def _merge_top8(a, b):
    d = [jnp.maximum(a[i], b[7 - i]) for i in range(8)]
    return _apply_net(d, _BITONIC8_STAGES)


def _topk_softmax_msg(s, P, g_raw, gamma, beta):
    ncol = s.shape[1] // 128
    sl = [s[:, k * 128:(k + 1) * 128] for k in range(ncol)]
    groups = [_apply_net(sl[8 * k:8 * k + 8], _SORT8_STAGES)
              for k in range(ncol // 8)]
    while len(groups) > 1:
        groups = [_merge_top8(groups[2 * k], groups[2 * k + 1])
                  for k in range(len(groups) // 2)]
    d = groups[0]
    m1 = jnp.max(d[0], axis=1, keepdims=True)
    m = m1
    for _ in range(TOPK - 1):
        eq = d[0] == m
        for i in range(TOPK - 1):
            d[i] = jnp.where(eq, d[i + 1], d[i])
        d[TOPK - 1] = jnp.where(eq, _NEG, d[TOPK - 1])
        m = jnp.max(d[0], axis=1, keepdims=True)
    thr = m
    e = jnp.where(s >= thr, jnp.exp(s - m1), 0.0)
    z = jnp.sum(e, axis=1, keepdims=True)
    msg = jax.lax.dot_general(
        e.astype(jnp.bfloat16), P,
        (((1,), (0,)), ((), ())),
        preferred_element_type=jnp.float32,
    ) / z
    return _layer_norm(g_raw + ALPHA * msg, gamma, beta)


def _body(gI_blk, gT_blk, gI_full, gT_full, W_i, W_t,
          ln_i_g, ln_i_b, ln_t_g, ln_t_b,
          S_ref, gI2_ref, gT2_ref,
          gin, gtn, Pi, Pt):
    i = pl.program_id(0)

    @pl.when(i == 0)
    def _init():
        gI = gI_full[...]
        gT = gT_full[...]
        gin[...] = _norm_rows(gI).astype(jnp.bfloat16)
        gtn[...] = _norm_rows(gT).astype(jnp.bfloat16)
        Pi[...] = jax.lax.dot_general(
            gI.astype(jnp.bfloat16), W_i[...].astype(jnp.bfloat16),
            (((1,), (1,)), ((), ())),
            preferred_element_type=jnp.float32).astype(jnp.bfloat16)
        Pt[...] = jax.lax.dot_general(
            gT.astype(jnp.bfloat16), W_t[...].astype(jnp.bfloat16),
            (((1,), (1,)), ((), ())),
            preferred_element_type=jnp.float32).astype(jnp.bfloat16)

    inv_tau = jnp.float32(1.0 / TAU)

    gib = _norm_rows(gI_blk[...]).astype(jnp.bfloat16)
    s = jax.lax.dot_general(
        gib, gtn[...],
        (((1,), (1,)), ((), ())),
        preferred_element_type=jnp.float32) * inv_tau
    S_ref[...] = s
    gI2_ref[...] = _topk_softmax_msg(
        s, Pt[...], gI_blk[...], ln_i_g[...], ln_i_b[...])

    gtb = _norm_rows(gT_blk[...]).astype(jnp.bfloat16)
    st = jax.lax.dot_general(
        gtb, gin[...],
        (((1,), (1,)), ((), ())),
        preferred_element_type=jnp.float32) * inv_tau
    gT2_ref[...] = _topk_softmax_msg(
        st, Pi[...], gT_blk[...], ln_t_g[...], ln_t_b[...])


@jax.jit
def kernel(gI, gT, W_i, W_t, ln_i_g, ln_i_b, ln_t_g, ln_t_b):
    grid = (B // BM,)
    blk = lambda i: (i, 0)
    full = lambda i: (0, 0)
    out_shapes = (
        jax.ShapeDtypeStruct((B, B), jnp.float32),
        jax.ShapeDtypeStruct((B, D), jnp.float32),
        jax.ShapeDtypeStruct((B, D), jnp.float32),
    )
    S, gI2, gT2 = pl.pallas_call(
        _body,
        grid=grid,
        in_specs=[
            pl.BlockSpec((BM, D), blk),
            pl.BlockSpec((BM, D), blk),
            pl.BlockSpec((B, D), full),
            pl.BlockSpec((B, D), full),
            pl.BlockSpec((D, D), full),
            pl.BlockSpec((D, D), full),
            pl.BlockSpec((1, D), full),
            pl.BlockSpec((1, D), full),
            pl.BlockSpec((1, D), full),
            pl.BlockSpec((1, D), full),
        ],
        out_specs=(
            pl.BlockSpec((BM, B), blk),
            pl.BlockSpec((BM, D), blk),
            pl.BlockSpec((BM, D), blk),
        ),
        out_shape=out_shapes,
        scratch_shapes=[
            pltpu.VMEM((B, D), jnp.bfloat16),
            pltpu.VMEM((B, D), jnp.bfloat16),
            pltpu.VMEM((B, D), jnp.bfloat16),
            pltpu.VMEM((B, D), jnp.bfloat16),
        ],
    )(gI, gT, gI, gT, W_i, W_t,
      ln_i_g.reshape(1, D), ln_i_b.reshape(1, D),
      ln_t_g.reshape(1, D), ln_t_b.reshape(1, D))
    return (gI2, gT2, S)


# speedup vs baseline: 15.0382x; 1.0346x over previous
import functools

import jax
import jax.numpy as jnp
from jax.experimental import pallas as pl
from jax.experimental.pallas import tpu as pltpu

TAU = 0.2
TOPK = 8
ALPHA = 0.3
B = 4096
D = 128
BM = 512

_NEG = -3.0e38


def _norm_rows(x):
    ss = jnp.sum(x * x, axis=1, keepdims=True)
    return x * jax.lax.rsqrt(jnp.maximum(ss, 1e-24))


def _layer_norm(y, gamma, beta):
    mu = jnp.mean(y, axis=1, keepdims=True)
    var = jnp.mean((y - mu) * (y - mu), axis=1, keepdims=True)
    return (y - mu) * jax.lax.rsqrt(var + 1e-5) * gamma + beta


_SORT8_STAGES = (
    ((0, 1), (2, 3), (4, 5), (6, 7)),
    ((0, 2), (1, 3), (4, 6), (5, 7)),
    ((1, 2), (5, 6)),
    ((0, 4), (1, 5), (2, 6), (3, 7)),
    ((2, 4), (3, 5)),
    ((1, 2), (3, 4), (5, 6)),
)
_BITONIC8_STAGES = (
    ((0, 4), (1, 5), (2, 6), (3, 7)),
    ((0, 2), (1, 3), (4, 6), (5, 7)),
    ((0, 1), (2, 3), (4, 5), (6, 7)),
)


def _apply_net(v, stages):
    v = list(v)
    for stage in stages:
        for i, j in stage:
            hi = jnp.maximum(v[i], v[j])
            lo = jnp.minimum(v[i], v[j])
            v[i], v[j] = hi, lo
    return v


def _merge_top8(a, b):
    d = [jnp.maximum(a[i], b[7 - i]) for i in range(8)]
    return _apply_net(d, _BITONIC8_STAGES)


def _topk_softmax_msg(s2, P, g_raw, gamma, beta):
    ncol = s2.shape[1] // 128
    sl = [s2[:, k * 128:(k + 1) * 128] for k in range(ncol)]
    groups = [_apply_net(sl[8 * k:8 * k + 8], _SORT8_STAGES)
              for k in range(ncol // 8)]
    while len(groups) > 1:
        groups = [_merge_top8(groups[2 * k], groups[2 * k + 1])
                  for k in range(len(groups) // 2)]
    d = groups[0]
    for _ in range(TOPK - 1):
        m = jnp.max(d[0], axis=1, keepdims=True)
        eq = d[0] == m
        for i in range(TOPK - 1):
            d[i] = jnp.where(eq, d[i + 1], d[i])
        d[TOPK - 1] = jnp.where(eq, _NEG, d[TOPK - 1])
    thr = jnp.max(d[0], axis=1, keepdims=True)
    e = jnp.where(s2 >= thr, jnp.exp2(s2), 0.0)
    z = jnp.sum(e, axis=1, keepdims=True)
    msg = jax.lax.dot_general(
        e.astype(jnp.bfloat16), P,
        (((1,), (0,)), ((), ())),
        preferred_element_type=jnp.float32,
    ) / z
    return _layer_norm(g_raw + ALPHA * msg, gamma, beta)


def _body(gI_blk, gT_blk, gI_full, gT_full, W_i, W_t,
          ln_i_g, ln_i_b, ln_t_g, ln_t_b,
          S_ref, gI2_ref, gT2_ref,
          gin, gtn, Pi, Pt):
    i = pl.program_id(0)

    c_scale = jnp.float32(1.4426950408889634 / TAU)

    @pl.when(i == 0)
    def _init():
        gI = gI_full[...]
        gT = gT_full[...]
        gin[...] = (_norm_rows(gI) * c_scale).astype(jnp.bfloat16)
        gtn[...] = (_norm_rows(gT) * c_scale).astype(jnp.bfloat16)
        Pi[...] = jax.lax.dot_general(
            gI.astype(jnp.bfloat16), W_i[...].astype(jnp.bfloat16),
            (((1,), (1,)), ((), ())),
            preferred_element_type=jnp.float32).astype(jnp.bfloat16)
        Pt[...] = jax.lax.dot_general(
            gT.astype(jnp.bfloat16), W_t[...].astype(jnp.bfloat16),
            (((1,), (1,)), ((), ())),
            preferred_element_type=jnp.float32).astype(jnp.bfloat16)

    ln2 = jnp.float32(0.6931471805599453)

    gib = _norm_rows(gI_blk[...]).astype(jnp.bfloat16)
    s2 = jax.lax.dot_general(
        gib, gtn[...],
        (((1,), (1,)), ((), ())),
        preferred_element_type=jnp.float32)
    S_ref[...] = s2 * ln2
    gI2_ref[...] = _topk_softmax_msg(
        s2, Pt[...], gI_blk[...], ln_i_g[...], ln_i_b[...])

    gtb = _norm_rows(gT_blk[...]).astype(jnp.bfloat16)
    st2 = jax.lax.dot_general(
        gtb, gin[...],
        (((1,), (1,)), ((), ())),
        preferred_element_type=jnp.float32)
    gT2_ref[...] = _topk_softmax_msg(
        st2, Pi[...], gT_blk[...], ln_t_g[...], ln_t_b[...])


@jax.jit
def kernel(gI, gT, W_i, W_t, ln_i_g, ln_i_b, ln_t_g, ln_t_b):
    grid = (B // BM,)
    blk = lambda i: (i, 0)
    full = lambda i: (0, 0)
    out_shapes = (
        jax.ShapeDtypeStruct((B, B), jnp.float32),
        jax.ShapeDtypeStruct((B, D), jnp.float32),
        jax.ShapeDtypeStruct((B, D), jnp.float32),
    )
    S, gI2, gT2 = pl.pallas_call(
        _body,
        grid=grid,
        in_specs=[
            pl.BlockSpec((BM, D), blk),
            pl.BlockSpec((BM, D), blk),
            pl.BlockSpec((B, D), full),
            pl.BlockSpec((B, D), full),
            pl.BlockSpec((D, D), full),
            pl.BlockSpec((D, D), full),
            pl.BlockSpec((1, D), full),
            pl.BlockSpec((1, D), full),
            pl.BlockSpec((1, D), full),
            pl.BlockSpec((1, D), full),
        ],
        out_specs=(
            pl.BlockSpec((BM, B), blk),
            pl.BlockSpec((BM, D), blk),
            pl.BlockSpec((BM, D), blk),
        ),
        out_shape=out_shapes,
        scratch_shapes=[
            pltpu.VMEM((B, D), jnp.bfloat16),
            pltpu.VMEM((B, D), jnp.bfloat16),
            pltpu.VMEM((B, D), jnp.bfloat16),
            pltpu.VMEM((B, D), jnp.bfloat16),
        ],
    )(gI, gT, gI, gT, W_i, W_t,
      ln_i_g.reshape(1, D), ln_i_b.reshape(1, D),
      ln_t_g.reshape(1, D), ln_t_b.reshape(1, D))
    return (gI2, gT2, S)
